# baseline (speedup 1.0000x reference)
import functools

import jax
import jax.numpy as jnp
from jax import lax
from jax.experimental import pallas as pl
from jax.experimental.pallas import tpu as pltpu
from jax.experimental.pallas import tpu_sc as plsc

_NC = 2
_NS = 16
_L = 16
_NW = _NC * _NS


def _tree_sum(vals):
    while len(vals) > 1:
        nxt = [vals[k] + vals[k + 1] for k in range(0, len(vals) - 1, 2)]
        if len(vals) % 2:
            nxt.append(vals[-1])
        vals = nxt
    return vals[0]


def _sc_combine_body(num_experts, d, tok_per_w, tchunk, dt,
                     eo_hbm, gates_hbm, out_hbm,
                     eo_v, gates_v, out_v0, out_v1, sem0, sem1):
    wid = lax.axis_index("c") * _NS + lax.axis_index("s")
    base = wid * tok_per_w
    ne = num_experts
    pltpu.sync_copy(eo_hbm, eo_v)
    pltpu.sync_copy(gates_hbm.at[pl.ds(base * ne, tok_per_w * ne)], gates_v)

    nvec = dt // _L
    bufs = (out_v0, out_v1)
    sems = (sem0, sem1)
    handles = [None, None]
    dnums = lax.GatherDimensionNumbers(
        offset_dims=(), collapsed_slice_dims=(0,), start_index_map=(0,))

    for tc in range(tok_per_w // tchunk):
        out_v = bufs[tc % 2]
        if handles[tc % 2] is not None:
            handles[tc % 2].wait()

        def dct_body(dct, _):
            dbase = pl.multiple_of(dct * dt, dt)
            eo_vecs = [[eo_v[e, pl.ds(dbase + j * _L, _L)] for j in range(nvec)]
                       for e in range(ne)]

            def pair_body(p, _):
                goff = pl.multiple_of((tc * tchunk + 2 * p) * ne, 2 * ne)
                grow = gates_v[pl.ds(goff, 2 * ne)]
                for parity in range(2):
                    g = [lax.gather(grow,
                                    jnp.full((_L, 1), parity * ne + e, jnp.int32),
                                    dnums, slice_sizes=(1,),
                                    mode=lax.GatherScatterMode.PROMISE_IN_BOUNDS)
                         for e in range(ne)]
                    for j in range(nvec):
                        prods = [g[e] * eo_vecs[e][j] for e in range(ne)]
                        out_v[2 * p + parity, pl.ds(dbase + j * _L, _L)] = (
                            _tree_sum(prods))
                return 0

            lax.fori_loop(0, tchunk // 2, pair_body, 0, unroll=2)
            return 0

        lax.fori_loop(0, d // dt, dct_body, 0)
        handles[tc % 2] = pltpu.async_copy(
            out_v, out_hbm.at[pl.ds(base + tc * tchunk, tchunk)], sems[tc % 2])
    for h in handles:
        h.wait()


def _sc_combine(expert_outputs, gates_flat, num_tokens, tchunk, num_cores=_NC):
    num_experts, d = expert_outputs.shape
    tok_per_w = num_tokens // (num_cores * _NS)
    dt = 64
    mesh = plsc.VectorSubcoreMesh(core_axis_name="c", subcore_axis_name="s",
                                  num_cores=num_cores)
    body = functools.partial(_sc_combine_body, num_experts, d, tok_per_w,
                             tchunk, dt)
    return pl.kernel(
        body,
        out_type=jax.ShapeDtypeStruct((num_tokens, d), jnp.float32),
        mesh=mesh,
        scratch_types=[
            pltpu.VMEM((num_experts, d), jnp.float32),
            pltpu.VMEM((tok_per_w * num_experts,), jnp.float32),
            pltpu.VMEM((tchunk, d), jnp.float32),
            pltpu.VMEM((tchunk, d), jnp.float32),
            pltpu.SemaphoreType.DMA,
            pltpu.SemaphoreType.DMA,
        ],
    )(expert_outputs, gates_flat)


def _tc_body(g_ref, e_ref, o_ref):
    o_ref[...] = jnp.dot(g_ref[...], e_ref[...],
                         preferred_element_type=jnp.float32)


def _tc_matmul(expert_outputs, gates, bm, out_rows):
    num_images, num_experts = gates.shape
    d = expert_outputs.shape[1]
    return pl.pallas_call(
        _tc_body,
        grid=(num_images // bm,),
        in_specs=[
            pl.BlockSpec((bm, num_experts), lambda i: (i, 0)),
            pl.BlockSpec((num_experts, d), lambda i: (0, 0)),
        ],
        out_specs=pl.BlockSpec((bm, d), lambda i: (i, 0)),
        out_shape=jax.ShapeDtypeStruct((out_rows, d), jnp.float32),
    )(gates, expert_outputs)


_SC_TOKENS = 512


def kernel(expert_outputs, gates):
    num_images, num_experts = gates.shape
    split = num_images - _SC_TOKENS
    out_sc = _sc_combine(
        expert_outputs,
        gates[split:].reshape(_SC_TOKENS * num_experts),
        _SC_TOKENS, tchunk=8, num_cores=1)
    out_tc = _tc_matmul(expert_outputs, gates[:split], bm=512,
                        out_rows=num_images)
    return lax.dynamic_update_slice(out_tc, out_sc, (split, 0))


# speedup vs baseline: 1.3828x; 1.3828x over previous
import functools

import jax
import jax.numpy as jnp
from jax import lax
from jax.experimental import pallas as pl
from jax.experimental.pallas import tpu as pltpu
from jax.experimental.pallas import tpu_sc as plsc

_NC = 2
_NS = 16
_L = 16
_NW = _NC * _NS


def _tree_sum(vals):
    while len(vals) > 1:
        nxt = [vals[k] + vals[k + 1] for k in range(0, len(vals) - 1, 2)]
        if len(vals) % 2:
            nxt.append(vals[-1])
        vals = nxt
    return vals[0]


def _sc_combine_body(num_experts, d, tok_per_w, tchunk, dt,
                     eo_hbm, gates_hbm, out_hbm,
                     eo_v, gates_v, out_v0, out_v1, sem0, sem1):
    wid = lax.axis_index("c") * _NS + lax.axis_index("s")
    base = wid * tok_per_w
    ne = num_experts
    pltpu.sync_copy(eo_hbm, eo_v)
    pltpu.sync_copy(gates_hbm.at[pl.ds(base * ne, tok_per_w * ne)], gates_v)

    nvec = dt // _L
    bufs = (out_v0, out_v1)
    sems = (sem0, sem1)
    handles = [None, None]
    dnums = lax.GatherDimensionNumbers(
        offset_dims=(), collapsed_slice_dims=(0,), start_index_map=(0,))

    for tc in range(tok_per_w // tchunk):
        out_v = bufs[tc % 2]
        if handles[tc % 2] is not None:
            handles[tc % 2].wait()

        def dct_body(dct, _):
            dbase = pl.multiple_of(dct * dt, dt)
            eo_vecs = [[eo_v[e, pl.ds(dbase + j * _L, _L)] for j in range(nvec)]
                       for e in range(ne)]

            def pair_body(p, _):
                goff = pl.multiple_of((tc * tchunk + 2 * p) * ne, 2 * ne)
                grow = gates_v[pl.ds(goff, 2 * ne)]
                for parity in range(2):
                    g = [lax.gather(grow,
                                    jnp.full((_L, 1), parity * ne + e, jnp.int32),
                                    dnums, slice_sizes=(1,),
                                    mode=lax.GatherScatterMode.PROMISE_IN_BOUNDS)
                         for e in range(ne)]
                    for j in range(nvec):
                        prods = [g[e] * eo_vecs[e][j] for e in range(ne)]
                        out_v[2 * p + parity, pl.ds(dbase + j * _L, _L)] = (
                            _tree_sum(prods))
                return 0

            lax.fori_loop(0, tchunk // 2, pair_body, 0, unroll=2)
            return 0

        lax.fori_loop(0, d // dt, dct_body, 0)
        handles[tc % 2] = pltpu.async_copy(
            out_v, out_hbm.at[pl.ds(base + tc * tchunk, tchunk)], sems[tc % 2])
    for h in handles:
        if h is not None:
            h.wait()


def _sc_combine(expert_outputs, gates_flat, num_tokens, tchunk, num_cores=_NC):
    num_experts, d = expert_outputs.shape
    tok_per_w = num_tokens // (num_cores * _NS)
    dt = 64
    mesh = plsc.VectorSubcoreMesh(core_axis_name="c", subcore_axis_name="s",
                                  num_cores=num_cores)
    body = functools.partial(_sc_combine_body, num_experts, d, tok_per_w,
                             tchunk, dt)
    return pl.kernel(
        body,
        out_type=jax.ShapeDtypeStruct((num_tokens, d), jnp.float32),
        mesh=mesh,
        scratch_types=[
            pltpu.VMEM((num_experts, d), jnp.float32),
            pltpu.VMEM((tok_per_w * num_experts,), jnp.float32),
            pltpu.VMEM((tchunk, d), jnp.float32),
            pltpu.VMEM((tchunk, d), jnp.float32),
            pltpu.SemaphoreType.DMA,
            pltpu.SemaphoreType.DMA,
        ],
    )(expert_outputs, gates_flat)


def _tc_body(g_ref, e_ref, o_ref):
    o_ref[...] = jnp.dot(g_ref[...], e_ref[...],
                         preferred_element_type=jnp.float32)


def _tc_matmul(expert_outputs, gates, bm, out_rows):
    num_images, num_experts = gates.shape
    d = expert_outputs.shape[1]
    return pl.pallas_call(
        _tc_body,
        grid=(num_images // bm,),
        in_specs=[
            pl.BlockSpec((bm, num_experts), lambda i: (i, 0)),
            pl.BlockSpec((num_experts, d), lambda i: (0, 0)),
        ],
        out_specs=pl.BlockSpec((bm, d), lambda i: (i, 0)),
        out_shape=jax.ShapeDtypeStruct((out_rows, d), jnp.float32),
    )(gates, expert_outputs)


_SC_TOKENS = 256


def kernel(expert_outputs, gates):
    num_images, num_experts = gates.shape
    split = num_images - _SC_TOKENS
    out_sc = _sc_combine(
        expert_outputs,
        gates[split:].reshape(_SC_TOKENS * num_experts),
        _SC_TOKENS, tchunk=8, num_cores=2)
    out_tc = _tc_matmul(expert_outputs, gates[:split], bm=512,
                        out_rows=num_images)
    return lax.dynamic_update_slice(out_tc, out_sc, (split, 0))
